# Initial kernel scaffold; baseline (speedup 1.0000x reference)
import jax
import jax.numpy as jnp
from jax.experimental import pallas as pl


def kernel(scores, viable, k):
    raise NotImplementedError("write your pallas kernel here")


# speedup vs baseline: 5.8105x; 5.8105x over previous
import functools

import jax
import jax.numpy as jnp
from jax import lax
from jax.experimental import pallas as pl
from jax.experimental.pallas import tpu as pltpu
from jax.experimental.pallas import tpu_sc as plsc

B, N, K = 64, 32768, 200
THRESHOLD = 0.05
L = 16
NCHUNKS = N // L
NWORDS = N // 32
INT_MIN = -(2**31)
IBIG = 2**31 - 1
K005 = 0x3D4CCCCD


def _splat(x, dtype=jnp.int32):
    return lax.broadcast(jnp.asarray(x, dtype), (L,))


def _zero_hist(hist, nchunks):
    zv = jnp.zeros((L,), jnp.int32)

    def zb(j, _):
        hist[pl.ds(j * L, L)] = zv
        return 0

    lax.fori_loop(0, nchunks, zb, 0)


def _select_level(hist, nchunks, kp):
    lane = lax.iota(jnp.int32, L)

    def body(j, carry):
        found, bstar, kpn, total = carry
        jj = nchunks - 1 - j
        h = hist[pl.ds(jj * L, L)]
        suf = lax.rev(plsc.cumsum(lax.rev(h, (0,))), (0,))
        ge = suf + _splat(total)
        csum = jnp.max(suf)
        kpv = _splat(kp)
        ncnt = jnp.sum(jnp.where(ge >= kpv, jnp.int32(1), jnp.int32(0)))
        hit = jnp.logical_and(found == 0, ncnt > 0)
        lstar = ncnt - 1
        hsel = jnp.max(jnp.where(lane == _splat(lstar), h, jnp.int32(0)))
        gesel = jnp.max(jnp.where(lane == _splat(lstar), ge, jnp.int32(0)))
        nb = jj * L + lstar
        nk = kp - (gesel - hsel)
        return (
            jnp.where(hit, jnp.int32(1), found),
            jnp.where(hit, nb, bstar),
            jnp.where(hit, nk, kpn),
            total + csum,
        )

    _, bstar, kpn, _ = lax.fori_loop(
        0, nchunks, body, (jnp.int32(0), jnp.int32(0), jnp.int32(0), jnp.int32(0))
    )
    return bstar, kpn


def _sc_body(scores_hbm, vmask_hbm, out_hbm, bs_hbm, bi_hbm, sbuf, mbuf, kbuf, hist, bsb, bib):
    info = plsc.get_sparse_core_info()
    nc = info.num_cores
    wid = lax.axis_index("s") * nc + lax.axis_index("c")
    lane = lax.iota(jnp.int32, L)
    onesv = jnp.ones((L,), jnp.int32)
    kK = jnp.int32(K)

    for rr in range(2):
        row = wid * 2 + rr
        pltpu.sync_copy(scores_hbm.at[row], sbuf)
        pltpu.sync_copy(vmask_hbm.at[row], mbuf)

        _zero_hist(hist, 128)

        def passA(gg, kmaxv):
            wv = mbuf[pl.ds(gg * L, L)]
            for th in range(2 * L):
                t, half = th // 2, th % 2
                w = _splat(wv[t])
                off = (gg * 2 * L + th) * L
                s = sbuf[pl.ds(off, L)]
                i = plsc.bitcast(s, jnp.int32)
                sk = lax.bitwise_xor(
                    i, lax.bitwise_and(lax.shift_right_arithmetic(i, _splat(31)), _splat(0x7FFFFFFF))
                )
                bit = lax.bitwise_and(
                    lax.shift_right_logical(w, lane + _splat(16 * half)), onesv
                )
                sk = jnp.where(bit > 0, sk, _splat(INT_MIN))
                kbuf[pl.ds(off, L)] = sk
                kmaxv = jnp.maximum(kmaxv, sk)
                bin3 = lax.bitwise_xor(lax.shift_right_logical(sk, _splat(21)), _splat(0x400))
                plsc.addupdate_scatter(hist, [bin3], onesv)
            return kmaxv

        kmaxv = lax.fori_loop(0, NWORDS // L, passA, _splat(INT_MIN))
        mkey = jnp.max(kmaxv)
        b3, k1 = _select_level(hist, 128, kK)

        _zero_hist(hist, 128)
        b3v = _splat(b3)
        mkeyv = _splat(mkey)

        def passB(c, idxminv):
            sk = kbuf[pl.ds(c * L, L)]
            hi11 = lax.bitwise_xor(lax.shift_right_logical(sk, _splat(21)), _splat(0x400))
            pm = hi11 == b3v
            bin2 = lax.bitwise_and(lax.shift_right_logical(sk, _splat(10)), _splat(0x7FF))
            plsc.addupdate_scatter(hist, [bin2], onesv, mask=pm)
            gidx = _splat(c * L) + lane
            idxminv = jnp.minimum(idxminv, jnp.where(sk == mkeyv, gidx, _splat(IBIG)))
            return idxminv

        idxminv = lax.fori_loop(0, NCHUNKS, passB, _splat(IBIG))
        bidx = jnp.min(idxminv)
        bidx = jnp.where(bidx == IBIG, jnp.int32(0), bidx)
        b2, k2 = _select_level(hist, 128, k1)

        _zero_hist(hist, 64)
        sel22 = jnp.left_shift(b3, jnp.int32(11)) | b2
        sel22v = _splat(sel22)

        def passC(c, _):
            sk = kbuf[pl.ds(c * L, L)]
            p22 = lax.bitwise_xor(lax.shift_right_logical(sk, _splat(10)), _splat(0x200000))
            pm = p22 == sel22v
            bin1 = lax.bitwise_and(sk, _splat(0x3FF))
            plsc.addupdate_scatter(hist, [bin1], onesv, mask=pm)
            return 0

        lax.fori_loop(0, NCHUNKS, passC, 0)
        b1, k3 = _select_level(hist, 64, k2)

        tkey = (
            jnp.left_shift(lax.bitwise_xor(b3, jnp.int32(0x400)), jnp.int32(21))
            | jnp.left_shift(b2, jnp.int32(10))
            | b1
        )
        tie_on = tkey >= K005
        tv = _splat(tkey)
        k005v = _splat(K005)
        rv = _splat(k3)

        def passE(c, cnt):
            sk = kbuf[pl.ds(c * L, L)]
            f = sbuf[pl.ds(c * L, L)]
            cond = jnp.logical_and(sk > tv, sk >= k005v)
            outv = jnp.where(cond, f, jnp.float32(0.0))
            eqi = jnp.where(sk == tv, jnp.int32(1), jnp.int32(0))
            neq = jnp.sum(eqi)

            def tie(args):
                outv, cnt = args
                pexc = plsc.cumsum(eqi) - eqi
                take = jnp.logical_and(eqi > 0, _splat(cnt) + pexc < rv)
                return jnp.where(take, f, outv), cnt + neq

            def notie(args):
                return args

            outv, cnt = lax.cond(
                jnp.logical_and(tie_on, neq > 0), tie, notie, (outv, cnt)
            )
            sbuf[pl.ds(c * L, L)] = outv
            return cnt

        lax.fori_loop(0, NCHUNKS, passE, jnp.int32(0))

        bsb[...] = plsc.load_gather(sbuf, [_splat(bidx)])
        bib[...] = _splat(bidx)
        pltpu.sync_copy(sbuf, out_hbm.at[row])
        pltpu.sync_copy(bsb, bs_hbm.at[row])
        pltpu.sync_copy(bib, bi_hbm.at[row])


@jax.jit
def _run(scores, vmask):
    mesh = plsc.VectorSubcoreMesh(core_axis_name="c", subcore_axis_name="s")
    fn = pl.kernel(
        _sc_body,
        out_type=[
            jax.ShapeDtypeStruct((B, N), jnp.float32),
            jax.ShapeDtypeStruct((B, L), jnp.float32),
            jax.ShapeDtypeStruct((B, L), jnp.int32),
        ],
        mesh=mesh,
        compiler_params=pltpu.CompilerParams(needs_layout_passes=False),
        scratch_types=[
            pltpu.VMEM((N,), jnp.float32),
            pltpu.VMEM((NWORDS,), jnp.int32),
            pltpu.VMEM((N,), jnp.int32),
            pltpu.VMEM((2048,), jnp.int32),
            pltpu.VMEM((L,), jnp.float32),
            pltpu.VMEM((L,), jnp.int32),
        ],
    )
    return fn(scores, vmask)


def kernel(scores, viable, k):
    bits = jnp.packbits(viable, axis=-1, bitorder="little")
    vmask = lax.bitcast_convert_type(bits.reshape(B, NWORDS, 4), jnp.int32)
    out, bs, bi = _run(scores, vmask)
    return out, bs[:, 0], bi[:, 0]


# speedup vs baseline: 6.1512x; 1.0586x over previous
import functools

import jax
import jax.numpy as jnp
from jax import lax
from jax.experimental import pallas as pl
from jax.experimental.pallas import tpu as pltpu
from jax.experimental.pallas import tpu_sc as plsc

B, N, K = 64, 32768, 200
THRESHOLD = 0.05
L = 16
NCHUNKS = N // L
NWORDS = N // 32
INT_MIN = -(2**31)
IBIG = 2**31 - 1
K005 = 0x3D4CCCCD


def _splat(x, dtype=jnp.int32):
    return lax.broadcast(jnp.asarray(x, dtype), (L,))


def _zero_hist(hist, nchunks):
    zv = jnp.zeros((L,), jnp.int32)

    def zb(j, _):
        hist[pl.ds(j * L, L)] = zv
        return 0

    lax.fori_loop(0, nchunks, zb, 0, unroll=8)


def _select_level(hist, nchunks, kp):
    lane = lax.iota(jnp.int32, L)

    def body(j, carry):
        found, bstar, kpn, esel, total = carry
        jj = nchunks - 1 - j
        h = hist[pl.ds(jj * L, L)]
        suf = lax.rev(plsc.cumsum(lax.rev(h, (0,))), (0,))
        ge = suf + _splat(total)
        csum = jnp.max(suf)
        kpv = _splat(kp)
        ncnt = jnp.sum(jnp.where(ge >= kpv, jnp.int32(1), jnp.int32(0)))
        hit = jnp.logical_and(found == 0, ncnt > 0)
        lstar = ncnt - 1
        hsel = jnp.max(jnp.where(lane == _splat(lstar), h, jnp.int32(0)))
        gesel = jnp.max(jnp.where(lane == _splat(lstar), ge, jnp.int32(0)))
        nb = jj * L + lstar
        nk = kp - (gesel - hsel)
        return (
            jnp.where(hit, jnp.int32(1), found),
            jnp.where(hit, nb, bstar),
            jnp.where(hit, nk, kpn),
            jnp.where(hit, hsel, esel),
            total + csum,
        )

    _, bstar, kpn, esel, _ = lax.fori_loop(
        0, nchunks, body,
        (jnp.int32(0), jnp.int32(0), jnp.int32(0), jnp.int32(0), jnp.int32(0)),
    )
    return bstar, kpn, esel


def _sc_body(scores_hbm, vmask_hbm, out_hbm, bs_hbm, bi_hbm, sbuf, mbuf, kbuf, hist, bsb, bib):
    info = plsc.get_sparse_core_info()
    nc = info.num_cores
    wid = lax.axis_index("s") * nc + lax.axis_index("c")
    lane = lax.iota(jnp.int32, L)
    onesv = jnp.ones((L,), jnp.int32)
    minv = _splat(INT_MIN)
    bm = [
        plsc.bitcast(jnp.left_shift(onesv, lane), jnp.int32),
        plsc.bitcast(jnp.left_shift(onesv, lane + _splat(16)), jnp.int32),
    ]
    kK = jnp.int32(K)

    for rr in range(2):
        row = wid * 2 + rr
        pltpu.sync_copy(scores_hbm.at[row], sbuf)
        pltpu.sync_copy(vmask_hbm.at[row], mbuf)

        _zero_hist(hist, 128)

        def passA(gg, carry):
            kmaxv, idxv, gidxv = carry
            wv = mbuf[pl.ds(gg * L, L)]
            for th in range(2 * L):
                t, half = th // 2, th % 2
                w = _splat(wv[t])
                off = (gg * 2 * L + th) * L
                s = sbuf[pl.ds(off, L)]
                i = plsc.bitcast(s, jnp.int32)
                sk = lax.bitwise_xor(
                    i,
                    lax.bitwise_and(
                        lax.shift_right_arithmetic(i, _splat(31)), _splat(0x7FFFFFFF)
                    ),
                )
                viab = lax.bitwise_and(w, bm[half]) != 0
                sk = jnp.where(viab, sk, minv)
                kbuf[pl.ds(off, L)] = sk
                gt = sk > kmaxv
                kmaxv = jnp.where(gt, sk, kmaxv)
                idxv = jnp.where(gt, gidxv, idxv)
                gidxv = gidxv + _splat(L)
                bin3 = lax.bitwise_xor(lax.shift_right_logical(sk, _splat(21)), _splat(0x400))
                plsc.addupdate_scatter(hist, [bin3], onesv)
            return kmaxv, idxv, gidxv

        kmaxv, idxv, _ = lax.fori_loop(
            0, NWORDS // L, passA, (minv, _splat(0), lane)
        )
        mkey = jnp.max(kmaxv)
        bidx = jnp.min(jnp.where(kmaxv == _splat(mkey), idxv, _splat(IBIG)))
        bidx = jnp.where(bidx == IBIG, jnp.int32(0), bidx)
        b3, k1, _ = _select_level(hist, 128, kK)

        _zero_hist(hist, 128)
        b3v = _splat(b3)

        def passB(c, _):
            sk = kbuf[pl.ds(c * L, L)]
            hi11 = lax.bitwise_xor(lax.shift_right_logical(sk, _splat(21)), _splat(0x400))
            bin2 = lax.bitwise_and(lax.shift_right_logical(sk, _splat(10)), _splat(0x7FF))
            plsc.addupdate_scatter(hist, [bin2], onesv, mask=hi11 == b3v)
            return 0

        lax.fori_loop(0, NCHUNKS, passB, 0, unroll=8)
        b2, k2, _ = _select_level(hist, 128, k1)

        _zero_hist(hist, 64)
        sel22v = _splat(jnp.left_shift(b3, jnp.int32(11)) | b2)

        def passC(c, _):
            sk = kbuf[pl.ds(c * L, L)]
            p22 = lax.bitwise_xor(lax.shift_right_logical(sk, _splat(10)), _splat(0x200000))
            bin1 = lax.bitwise_and(sk, _splat(0x3FF))
            plsc.addupdate_scatter(hist, [bin1], onesv, mask=p22 == sel22v)
            return 0

        lax.fori_loop(0, NCHUNKS, passC, 0, unroll=8)
        b1, k3, ecnt = _select_level(hist, 64, k2)

        tkey = (
            jnp.left_shift(lax.bitwise_xor(b3, jnp.int32(0x400)), jnp.int32(21))
            | jnp.left_shift(b2, jnp.int32(10))
            | b1
        )
        tie_on = tkey >= jnp.int32(K005)
        tv = _splat(tkey)
        k005v = _splat(K005)
        rv = _splat(k3)

        zf = jnp.zeros((L,), jnp.float32)

        def passE(c, _):
            sk = kbuf[pl.ds(c * L, L)]
            f = sbuf[pl.ds(c * L, L)]
            keep = jnp.logical_and(sk >= tv, sk >= k005v)
            sbuf[pl.ds(c * L, L)] = jnp.where(keep, f, zf)
            return 0

        lax.fori_loop(0, NCHUNKS, passE, 0, unroll=8)

        def fix(_):
            def fbody(c, cntv):
                sk = kbuf[pl.ds(c * L, L)]
                eq = sk == tv
                eqi = jnp.where(eq, jnp.int32(1), jnp.int32(0))
                pexc = plsc.cumsum(eqi) - eqi
                drop = jnp.logical_and(eq, cntv + pexc >= rv)
                o = sbuf[pl.ds(c * L, L)]
                sbuf[pl.ds(c * L, L)] = jnp.where(drop, zf, o)
                return cntv + plsc.all_reduce_population_count(eq)

            lax.fori_loop(0, NCHUNKS, fbody, _splat(0))
            return 0

        lax.cond(jnp.logical_and(tie_on, ecnt > k3), fix, lambda _: 0, 0)

        bsb[...] = plsc.load_gather(sbuf, [_splat(bidx)])
        bib[...] = _splat(bidx)
        pltpu.sync_copy(sbuf, out_hbm.at[row])
        pltpu.sync_copy(bsb, bs_hbm.at[row])
        pltpu.sync_copy(bib, bi_hbm.at[row])


@jax.jit
def _run(scores, vmask):
    mesh = plsc.VectorSubcoreMesh(core_axis_name="c", subcore_axis_name="s")
    fn = pl.kernel(
        _sc_body,
        out_type=[
            jax.ShapeDtypeStruct((B, N), jnp.float32),
            jax.ShapeDtypeStruct((B, L), jnp.float32),
            jax.ShapeDtypeStruct((B, L), jnp.int32),
        ],
        mesh=mesh,
        compiler_params=pltpu.CompilerParams(needs_layout_passes=False),
        scratch_types=[
            pltpu.VMEM((N,), jnp.float32),
            pltpu.VMEM((NWORDS,), jnp.int32),
            pltpu.VMEM((N,), jnp.int32),
            pltpu.VMEM((2048,), jnp.int32),
            pltpu.VMEM((L,), jnp.float32),
            pltpu.VMEM((L,), jnp.int32),
        ],
    )
    return fn(scores, vmask)


def kernel(scores, viable, k):
    bits = jnp.packbits(viable, axis=-1, bitorder="little")
    vmask = lax.bitcast_convert_type(bits.reshape(B, NWORDS, 4), jnp.int32)
    out, bs, bi = _run(scores, vmask)
    return out, bs[:, 0], bi[:, 0]


# speedup vs baseline: 8.7805x; 1.4275x over previous
import functools

import jax
import jax.numpy as jnp
from jax import lax
from jax.experimental import pallas as pl
from jax.experimental.pallas import tpu as pltpu
from jax.experimental.pallas import tpu_sc as plsc

B, N, K = 64, 32768, 200
THRESHOLD = 0.05
L = 16
NCHUNKS = N // L
NWORDS = N // 32
INT_MIN = -(2**31)
IBIG = 2**31 - 1
K005 = 0x3D4CCCCD


def _splat(x, dtype=jnp.int32):
    return lax.broadcast(jnp.asarray(x, dtype), (L,))


def _zero_hist(hist, nchunks):
    zv = jnp.zeros((L,), jnp.int32)

    def zb(j, _):
        hist[pl.ds(j * L, L)] = zv
        return 0

    lax.fori_loop(0, nchunks, zb, 0, unroll=8)


def _select_level(hist, nchunks, kp):
    lane = lax.iota(jnp.int32, L)

    def body(j, carry):
        found, bstar, kpn, esel, total = carry
        jj = nchunks - 1 - j
        h = hist[pl.ds(jj * L, L)]
        suf = lax.rev(plsc.cumsum(lax.rev(h, (0,))), (0,))
        ge = suf + _splat(total)
        csum = jnp.max(suf)
        kpv = _splat(kp)
        ncnt = jnp.sum(jnp.where(ge >= kpv, jnp.int32(1), jnp.int32(0)))
        hit = jnp.logical_and(found == 0, ncnt > 0)
        lstar = ncnt - 1
        hsel = jnp.max(jnp.where(lane == _splat(lstar), h, jnp.int32(0)))
        gesel = jnp.max(jnp.where(lane == _splat(lstar), ge, jnp.int32(0)))
        nb = jj * L + lstar
        nk = kp - (gesel - hsel)
        return (
            jnp.where(hit, jnp.int32(1), found),
            jnp.where(hit, nb, bstar),
            jnp.where(hit, nk, kpn),
            jnp.where(hit, hsel, esel),
            total + csum,
        )

    found, bstar, kpn, esel, _ = lax.fori_loop(
        0, nchunks, body,
        (jnp.int32(0), jnp.int32(0), jnp.int32(0), jnp.int32(0), jnp.int32(0)),
    )
    return bstar, kpn, esel, found


def _sc_body(
    scores_hbm, vmask_hbm, out_hbm, bs_hbm, bi_hbm, sbuf, mbuf, kbuf, cand, hist, bsb, bib
):
    info = plsc.get_sparse_core_info()
    nc = info.num_cores
    wid = lax.axis_index("s") * nc + lax.axis_index("c")
    lane = lax.iota(jnp.int32, L)
    onesv = jnp.ones((L,), jnp.int32)
    minv = _splat(INT_MIN)
    k005v = _splat(K005)
    bmask = [
        plsc.bitcast(jnp.left_shift(onesv, lane), jnp.int32),
        plsc.bitcast(jnp.left_shift(onesv, lane + _splat(16)), jnp.int32),
    ]
    kK = jnp.int32(K)

    for rr in range(2):
        row = wid * 2 + rr
        pltpu.sync_copy(scores_hbm.at[row], sbuf)
        pltpu.sync_copy(vmask_hbm.at[row], mbuf)

        _zero_hist(hist, 128)

        def passA(gg, carry):
            kmaxv, idxv, gidxv, off = carry
            wv = mbuf[pl.ds(gg * L, L)]
            for th in range(2 * L):
                t, half = th // 2, th % 2
                w = _splat(wv[t])
                base = (gg * 2 * L + th) * L
                s = sbuf[pl.ds(base, L)]
                i = plsc.bitcast(s, jnp.int32)
                sk = lax.bitwise_xor(
                    i,
                    lax.bitwise_and(
                        lax.shift_right_arithmetic(i, _splat(31)), _splat(0x7FFFFFFF)
                    ),
                )
                viab = lax.bitwise_and(w, bmask[half]) != 0
                sk = jnp.where(viab, sk, minv)
                kbuf[pl.ds(base, L)] = sk
                gt = sk > kmaxv
                kmaxv = jnp.where(gt, sk, kmaxv)
                idxv = jnp.where(gt, gidxv, idxv)
                gidxv = gidxv + _splat(L)
                m005 = sk >= k005v
                plsc.store_compressed(cand.at[pl.ds(off, L)], sk, mask=m005)
                off = off + plsc.all_reduce_population_count(m005)[0]
                bin3 = lax.bitwise_xor(
                    lax.shift_right_logical(sk, _splat(21)), _splat(0x400)
                )
                plsc.addupdate_scatter(hist, [bin3], onesv, mask=m005)
            return kmaxv, idxv, gidxv, off

        kmaxv, idxv, _, noff = lax.fori_loop(
            0, NWORDS // L, passA, (minv, _splat(0), lane, jnp.int32(0))
        )
        mkey = jnp.max(kmaxv)
        bidx = jnp.min(jnp.where(kmaxv == _splat(mkey), idxv, _splat(IBIG)))
        bidx = jnp.where(bidx == IBIG, jnp.int32(0), bidx)
        cand[pl.ds(noff, L)] = minv
        candc = lax.shift_right_logical(noff + jnp.int32(L - 1), jnp.int32(4))
        b3, k1, _, found = _select_level(hist, 128, kK)

        _zero_hist(hist, 128)
        b3v = _splat(b3)

        def passB(c, _):
            sk = cand[pl.ds(c * L, L)]
            hi11 = lax.bitwise_xor(lax.shift_right_logical(sk, _splat(21)), _splat(0x400))
            bin2 = lax.bitwise_and(lax.shift_right_logical(sk, _splat(10)), _splat(0x7FF))
            plsc.addupdate_scatter(hist, [bin2], onesv, mask=hi11 == b3v)
            return 0

        lax.fori_loop(0, candc, passB, 0)
        b2, k2, _, _ = _select_level(hist, 128, k1)

        _zero_hist(hist, 64)
        sel22v = _splat(jnp.left_shift(b3, jnp.int32(11)) | b2)

        def passC(c, _):
            sk = cand[pl.ds(c * L, L)]
            p22 = lax.bitwise_xor(lax.shift_right_logical(sk, _splat(10)), _splat(0x200000))
            bin1 = lax.bitwise_and(sk, _splat(0x3FF))
            plsc.addupdate_scatter(hist, [bin1], onesv, mask=p22 == sel22v)
            return 0

        lax.fori_loop(0, candc, passC, 0)
        b1, k3, ecnt, _ = _select_level(hist, 64, k2)

        tkey = (
            jnp.left_shift(lax.bitwise_xor(b3, jnp.int32(0x400)), jnp.int32(21))
            | jnp.left_shift(b2, jnp.int32(10))
            | b1
        )
        tkey = jnp.where(found > 0, tkey, jnp.int32(K005))
        tv = _splat(tkey)
        rv = _splat(k3)

        zf = jnp.zeros((L,), jnp.float32)

        def passE(c, _):
            sk = kbuf[pl.ds(c * L, L)]
            f = sbuf[pl.ds(c * L, L)]
            sbuf[pl.ds(c * L, L)] = jnp.where(sk >= tv, f, zf)
            return 0

        lax.fori_loop(0, NCHUNKS, passE, 0, unroll=8)

        def fix(_):
            def fbody(c, cntv):
                sk = kbuf[pl.ds(c * L, L)]
                eq = sk == tv
                eqi = jnp.where(eq, jnp.int32(1), jnp.int32(0))
                pexc = plsc.cumsum(eqi) - eqi
                drop = jnp.logical_and(eq, cntv + pexc >= rv)
                o = sbuf[pl.ds(c * L, L)]
                sbuf[pl.ds(c * L, L)] = jnp.where(drop, zf, o)
                return cntv + plsc.all_reduce_population_count(eq)

            lax.fori_loop(0, NCHUNKS, fbody, _splat(0))
            return 0

        lax.cond(jnp.logical_and(found > 0, ecnt > k3), fix, lambda _: 0, 0)

        bsb[...] = plsc.load_gather(sbuf, [_splat(bidx)])
        bib[...] = _splat(bidx)
        pltpu.sync_copy(sbuf, out_hbm.at[row])
        pltpu.sync_copy(bsb, bs_hbm.at[row])
        pltpu.sync_copy(bib, bi_hbm.at[row])


@jax.jit
def _run(scores, vmask):
    mesh = plsc.VectorSubcoreMesh(core_axis_name="c", subcore_axis_name="s")
    fn = pl.kernel(
        _sc_body,
        out_type=[
            jax.ShapeDtypeStruct((B, N), jnp.float32),
            jax.ShapeDtypeStruct((B, L), jnp.float32),
            jax.ShapeDtypeStruct((B, L), jnp.int32),
        ],
        mesh=mesh,
        compiler_params=pltpu.CompilerParams(needs_layout_passes=False),
        scratch_types=[
            pltpu.VMEM((N,), jnp.float32),
            pltpu.VMEM((NWORDS,), jnp.int32),
            pltpu.VMEM((N,), jnp.int32),
            pltpu.VMEM((N + L,), jnp.int32),
            pltpu.VMEM((2048,), jnp.int32),
            pltpu.VMEM((L,), jnp.float32),
            pltpu.VMEM((L,), jnp.int32),
        ],
    )
    return fn(scores, vmask)


def kernel(scores, viable, k):
    bits = jnp.packbits(viable, axis=-1, bitorder="little")
    vmask = lax.bitcast_convert_type(bits.reshape(B, NWORDS, 4), jnp.int32)
    out, bs, bi = _run(scores, vmask)
    return out, bs[:, 0], bi[:, 0]
